# baseline (speedup 1.0000x reference)
import functools

import jax
import jax.numpy as jnp
from jax import lax
from jax.experimental import pallas as pl
from jax.experimental.pallas import tpu as pltpu
from jax.experimental.pallas import tpu_sc as plsc

L = 16


def kernel(u_idx, i_idx, u_emb, i_emb, u_bias, i_bias):
    B = u_idx.shape[0]
    N, F = u_emb.shape
    info = plsc.get_sparse_core_info()
    NC, NS = info.num_cores, info.num_subcores
    NW = NC * NS
    b_per_w = B // NW

    mesh = plsc.VectorSubcoreMesh(core_axis_name="c", subcore_axis_name="s")

    @functools.partial(
        pl.kernel,
        mesh=mesh,
        out_type=jax.ShapeDtypeStruct((B,), jnp.float32),
        compiler_params=pltpu.CompilerParams(
            needs_layout_passes=False, use_tc_tiling_on_sc=False),
        scratch_types=[
            pltpu.VMEM((b_per_w,), jnp.int32),
            pltpu.VMEM((b_per_w,), jnp.int32),
            pltpu.VMEM((F, b_per_w), jnp.float32),
            pltpu.VMEM((F, b_per_w), jnp.float32),
            pltpu.VMEM((b_per_w,), jnp.float32),
            pltpu.VMEM((b_per_w,), jnp.float32),
            pltpu.VMEM((b_per_w,), jnp.float32),
            pltpu.SemaphoreType.DMA,
        ],
    )
    def sc_kernel(u_idx_hbm, i_idx_hbm, ut_hbm, it_hbm, ubt_hbm, ibt_hbm,
                  out_hbm, uidx_v, iidx_v, ubuf, ibuf, ubv, ibv, out_v, sem):
        wid = lax.axis_index("s") * NC + lax.axis_index("c")
        base = wid * b_per_w
        pltpu.sync_copy(u_idx_hbm.at[pl.ds(base, b_per_w)], uidx_v)
        pltpu.sync_copy(i_idx_hbm.at[pl.ds(base, b_per_w)], iidx_v)

        for f in range(F):
            pltpu.async_copy(ut_hbm.at[f].at[uidx_v], ubuf.at[f], sem)
            pltpu.async_copy(it_hbm.at[f].at[iidx_v], ibuf.at[f], sem)
        cub = pltpu.async_copy(ubt_hbm.at[0].at[uidx_v], ubv, sem)
        cib = pltpu.async_copy(ibt_hbm.at[0].at[iidx_v], ibv, sem)
        for f in range(F):
            pltpu.make_async_copy(ut_hbm.at[f].at[pl.ds(0, b_per_w)],
                                  ubuf.at[f], sem).wait()
            pltpu.make_async_copy(it_hbm.at[f].at[pl.ds(0, b_per_w)],
                                  ibuf.at[f], sem).wait()
        cub.wait()
        cib.wait()

        def body(g, carry):
            acc = ubv[pl.ds(g * L, L)] + ibv[pl.ds(g * L, L)]
            for f in range(F):
                acc = acc + (ubuf[f, pl.ds(g * L, L)]
                             * ibuf[f, pl.ds(g * L, L)])
            out_v[pl.ds(g * L, L)] = acc
            return carry

        lax.fori_loop(0, b_per_w // L, body, 0)
        pltpu.sync_copy(out_v, out_hbm.at[pl.ds(base, b_per_w)])

    return sc_kernel(u_idx, i_idx, u_emb.T, i_emb.T, u_bias.T, i_bias.T)


# speedup vs baseline: 5.9322x; 5.9322x over previous
import functools

import jax
import jax.numpy as jnp
from jax import lax
from jax.experimental import pallas as pl
from jax.experimental.pallas import tpu as pltpu
from jax.experimental.pallas import tpu_sc as plsc

L = 16


def kernel(u_idx, i_idx, u_emb, i_emb, u_bias, i_bias):
    B = u_idx.shape[0]
    N, F = u_emb.shape
    info = plsc.get_sparse_core_info()
    NC, NS = info.num_cores, info.num_subcores
    NW = NC * NS
    b_per_w = B // NW

    mesh = plsc.VectorSubcoreMesh(core_axis_name="c", subcore_axis_name="s")

    @functools.partial(
        pl.kernel,
        mesh=mesh,
        out_type=jax.ShapeDtypeStruct((B,), jnp.float32),
        compiler_params=pltpu.CompilerParams(
            needs_layout_passes=False, use_tc_tiling_on_sc=False),
        scratch_types=[
            pltpu.VMEM((b_per_w,), jnp.int32),
            pltpu.VMEM((b_per_w,), jnp.int32),
            pltpu.VMEM((b_per_w, F), jnp.float32),
            pltpu.VMEM((b_per_w, F), jnp.float32),
            pltpu.VMEM((b_per_w,), jnp.float32),
            pltpu.VMEM((b_per_w,), jnp.float32),
            pltpu.VMEM((b_per_w,), jnp.float32),
            pltpu.SemaphoreType.DMA,
        ],
    )
    def sc_kernel(u_idx_hbm, i_idx_hbm, ue_hbm, ie_hbm, ubt_hbm, ibt_hbm,
                  out_hbm, uidx_v, iidx_v, urows_v, irows_v, ubv, ibv,
                  out_v, sem):
        wid = lax.axis_index("s") * NC + lax.axis_index("c")
        base = wid * b_per_w
        pltpu.sync_copy(u_idx_hbm.at[pl.ds(base, b_per_w)], uidx_v)
        pltpu.sync_copy(i_idx_hbm.at[pl.ds(base, b_per_w)], iidx_v)
        cu = pltpu.async_copy(ue_hbm.at[uidx_v], urows_v, sem)
        ci = pltpu.async_copy(ie_hbm.at[iidx_v], irows_v, sem)
        cub = pltpu.async_copy(ubt_hbm.at[0].at[uidx_v], ubv, sem)
        cib = pltpu.async_copy(ibt_hbm.at[0].at[iidx_v], ibv, sem)
        cu.wait()
        ci.wait()
        cub.wait()
        cib.wait()

        def body(g, carry):
            rows = lax.iota(jnp.int32, L) + g * L
            acc = ubv[pl.ds(g * L, L)] + ibv[pl.ds(g * L, L)]
            for f in range(F):
                cols = jnp.full((L,), f, jnp.int32)
                uv = plsc.load_gather(urows_v, [rows, cols])
                iv = plsc.load_gather(irows_v, [rows, cols])
                acc = acc + uv * iv
            out_v[pl.ds(g * L, L)] = acc
            return carry

        lax.fori_loop(0, b_per_w // L, body, 0)
        pltpu.sync_copy(out_v, out_hbm.at[pl.ds(base, b_per_w)])

    return sc_kernel(u_idx, i_idx, u_emb, i_emb, u_bias.T, i_bias.T)
